# baseline (speedup 1.0000x reference)
import jax
import jax.numpy as jnp
from jax import lax
from jax.experimental import pallas as pl
from jax.experimental.pallas import tpu as pltpu
from jax.experimental.pallas import tpu_sc as plsc

NC_CLS = 80
VFL_A = 0.75
M_ALPHA = 0.25
W_BBOX_C, W_CLASS_C, W_GIOU_C = 5.0, 2.0, 2.0
W_VFL_L, W_BBOX_L, W_GIOU_L = 1.0, 5.0, 2.0
B, NQ, NT = 16, 300, 32
MPAD = 304
NCH = MPAD // 16
INF = 1e18
PADVAL = 1e9


def _sigmoid(x):
    return 1.0 / (1.0 + jnp.exp(-x))


def _xyxy(bT):
    cx, cy, w, h = bT[0:1, :], bT[1:2, :], bT[2:3, :], bT[3:4, :]
    return cx - 0.5 * w, cy - 0.5 * h, cx + 0.5 * w, cy + 0.5 * h


IMS = 4


def _cost_dense_body(logits_ref, pb_ref, lab_ref, tb_ref, cost_ref, dense_ref):
    b = pl.program_id(0)
    dsum = jnp.float32(0.0)
    for im in range(IMS):
        x = logits_ref[im]
        prob = _sigmoid(x)
        neg_cost = (1.0 - M_ALPHA) * (prob * prob) * (-jnp.log(1.0 - prob + 1e-8))
        pos_cost = M_ALPHA * ((1.0 - prob) * (1.0 - prob)) * (-jnp.log(prob + 1e-8))
        diff = pos_cost - neg_cost

        lab = lab_ref[im, 0, :]
        cls_iota = lax.broadcasted_iota(jnp.int32, (NT, NC_CLS), 1)
        onehot = (lab[:, None] == cls_iota).astype(jnp.float32)
        cost_class = lax.dot_general(
            onehot, diff, (((1,), (1,)), ((), ())),
            preferred_element_type=jnp.float32)

        pbT = pb_ref[im]
        tbT = tb_ref[im]
        cost_bbox = jnp.zeros((NT, NQ), jnp.float32)
        for k in range(4):
            cost_bbox = cost_bbox + jnp.abs(tbT[k:k + 1, :].T - pbT[k:k + 1, :])

        tcx, tcy, tw, th = (tbT[0:1, :].T, tbT[1:2, :].T,
                            tbT[2:3, :].T, tbT[3:4, :].T)
        tx0, ty0 = tcx - 0.5 * tw, tcy - 0.5 * th
        tx1, ty1 = tcx + 0.5 * tw, tcy + 0.5 * th
        px0, py0, px1, py1 = _xyxy(pbT)
        area_t = (tx1 - tx0) * (ty1 - ty0)
        area_p = (px1 - px0) * (py1 - py0)
        iw = jnp.clip(jnp.minimum(tx1, px1) - jnp.maximum(tx0, px0), 0.0)
        ih = jnp.clip(jnp.minimum(ty1, py1) - jnp.maximum(ty0, py0), 0.0)
        inter = iw * ih
        union = area_t + area_p - inter
        iou = inter / union
        ew = jnp.clip(jnp.maximum(tx1, px1) - jnp.minimum(tx0, px0), 0.0)
        eh = jnp.clip(jnp.maximum(ty1, py1) - jnp.minimum(ty0, py0), 0.0)
        earea = ew * eh
        giou = iou - (earea - union) / earea

        cmat = W_BBOX_C * cost_bbox + W_CLASS_C * cost_class + W_GIOU_C * (-giou)
        cost_ref[im] = jnp.concatenate(
            [cmat, jnp.full((NT, MPAD - NQ), PADVAL, jnp.float32)], axis=1)

        bce0 = jnp.maximum(x, 0.0) + jnp.log1p(jnp.exp(-jnp.abs(x)))
        dsum = dsum + jnp.sum(VFL_A * prob * prob * bce0)

    lane = lax.broadcasted_iota(jnp.int32, (1, 128), 1)
    dvec = jnp.where(lane == 0, dsum, 0.0)

    @pl.when(b == 0)
    def _():
        dense_ref[...] = jnp.zeros_like(dense_ref)

    dense_ref[...] += dvec


def _sc_solver_body(cost_hbm, out_hbm, cost_v, v_v, w_v, minv_v, way_v,
                    used_v, asn_v, p_s):
    wid = lax.axis_index("s") * 2 + lax.axis_index("c")

    @pl.when(wid < B)
    def _():
        pltpu.sync_copy(cost_hbm.at[wid], cost_v)
        iota16 = lax.broadcasted_iota(jnp.int32, (16,), 0)
        zf = jnp.zeros((16,), jnp.float32)
        zi = jnp.zeros((16,), jnp.int32)
        for k in range(NCH):
            sl = pl.ds(16 * k, 16)
            v_v[sl] = zf
            w_v[sl] = zf
            way_v[sl] = zi

        def clear_p(j, c):
            p_s[j] = jnp.int32(0)
            return c

        lax.fori_loop(0, NQ + 1, clear_p, 0)

        def phase(i, c):
            used_init = ([jnp.zeros((16,), jnp.int32)] * (NCH - 1)
                         + [(iota16 >= (16 - (MPAD - NQ))).astype(jnp.int32)])
            p_s[0] = i
            for k in range(NCH):
                sl = pl.ds(16 * k, 16)
                minv_v[sl] = jnp.full((16,), INF, jnp.float32)
                used_v[sl] = used_init[k]

            def cond(carry):
                return carry[2] != 0

            def body(carry):
                j0, w0, pj0 = carry
                jj0 = jnp.maximum(j0 - 1, 0)
                jj0_splat = jnp.full((16,), jj0, jnp.int32)
                lane0 = iota16 == 0
                j0_pos = jnp.full((16,), j0, jnp.int32) > 0
                plsc.store_scatter(used_v, [jj0_splat],
                                   jnp.ones((16,), jnp.int32),
                                   mask=lane0 & j0_pos)

                w_at_jj0 = plsc.load_gather(w_v, [jj0_splat])
                w0_splat = jnp.full((16,), w0, jnp.float32)
                u_i0 = jnp.where(j0_pos, w_at_jj0, w0_splat)
                row_splat = jnp.full((16,), pj0 - 1, jnp.int32)
                j0_splat = jnp.full((16,), j0, jnp.int32)
                bestv = jnp.full((16,), INF, jnp.float32)
                bestj = jnp.zeros((16,), jnp.int32)
                for k in range(NCH):
                    sl = pl.ds(16 * k, 16)
                    crow = plsc.load_gather(cost_v, [row_splat, iota16 + 16 * k])
                    cu = (crow - u_i0) - v_v[sl]
                    freek = used_v[sl] == 0
                    mv = minv_v[sl]
                    upd = freek & (cu < mv)
                    mv = jnp.where(upd, cu, mv)
                    minv_v[sl] = mv
                    way_v[sl] = jnp.where(upd, j0_splat, way_v[sl])
                    masked = jnp.where(freek, mv, INF)
                    better = masked < bestv
                    bestv = jnp.where(better, masked, bestv)
                    bestj = jnp.where(better, iota16 + (16 * k + 1), bestj)
                delta = jnp.min(bestv)
                deltav = jnp.full((16,), delta, jnp.float32)
                zerov = jnp.zeros((16,), jnp.float32)
                j1 = jnp.min(jnp.where(bestv == deltav, bestj,
                                       jnp.full((16,), 1 << 30, jnp.int32)))
                for k in range(NCH):
                    sl = pl.ds(16 * k, 16)
                    um = used_v[sl] != 0
                    dmask = jnp.where(um, deltav, zerov)
                    w_v[sl] = w_v[sl] + dmask
                    v_v[sl] = v_v[sl] - dmask
                    minv_v[sl] = jnp.where(um, minv_v[sl], minv_v[sl] - deltav)
                return (j1, w0 + delta, p_s[j1])

            j0f, w0f, _ = lax.while_loop(
                cond, body, (jnp.int32(0), jnp.float32(0.0), i))

            lane0 = iota16 == 0

            def abody(j0):
                jj0_splat = jnp.full((16,), j0 - 1, jnp.int32)
                j1 = jnp.min(plsc.load_gather(way_v, [jj0_splat]))
                row = p_s[j1]
                p_s[j0] = row
                plsc.store_scatter(asn_v, [jnp.full((16,), row - 1, jnp.int32)],
                                   jnp.full((16,), j0 - 1, jnp.int32),
                                   mask=lane0)
                jj1_splat = jnp.full((16,), jnp.maximum(j1 - 1, 0), jnp.int32)
                wsrc = jnp.min(plsc.load_gather(w_v, [jj1_splat]))
                wnew = jnp.where(j1 == 0, w0f, wsrc)
                plsc.store_scatter(w_v, [jj0_splat],
                                   jnp.full((16,), wnew, jnp.float32),
                                   mask=lane0)
                return j1

            lax.while_loop(lambda j0: j0 != 0, abody, j0f)
            return c

        lax.fori_loop(1, NT + 1, phase, 0)
        pltpu.sync_copy(asn_v, out_hbm.at[wid])


def _corr_body(logits_ref, pb_ref, lab_ref, tb_ref, asn_ref, acc_ref):
    b = pl.program_id(0)
    corr_sum = jnp.float32(0.0)
    bbox_sum = jnp.float32(0.0)
    giou_sum = jnp.float32(0.0)
    for im in range(IMS):
        x = logits_ref[im]
        q = asn_ref[im, 0, :]
        lab = lab_ref[im, 0, :]

        q_iota = lax.broadcasted_iota(jnp.int32, (NT, NQ), 1)
        ohq = (q[:, None] == q_iota).astype(jnp.float32)
        xl_rows = lax.dot_general(
            ohq, x, (((1,), (0,)), ((), ())),
            preferred_element_type=jnp.float32)
        cls_iota = lax.broadcasted_iota(jnp.int32, (NT, NC_CLS), 1)
        ohl = (lab[:, None] == cls_iota).astype(jnp.float32)
        xl = jnp.sum(xl_rows * ohl, axis=1)[:, None]

        pbT = pb_ref[im]
        tbT = tb_ref[im]
        sbT = lax.dot_general(
            pbT, ohq, (((1,), (1,)), ((), ())),
            preferred_element_type=jnp.float32)
        sc_, sy_, sw_, sh_ = (sbT[0:1, :].T, sbT[1:2, :].T,
                              sbT[2:3, :].T, sbT[3:4, :].T)
        tc_, tyc, tw_, th_ = (tbT[0:1, :].T, tbT[1:2, :].T,
                              tbT[2:3, :].T, tbT[3:4, :].T)

        sx0 = sc_ - 0.5 * sw_
        sy0 = sy_ - 0.5 * sh_
        sx1 = sc_ + 0.5 * sw_
        sy1 = sy_ + 0.5 * sh_
        tx0 = tc_ - 0.5 * tw_
        ty0 = tyc - 0.5 * th_
        tx1 = tc_ + 0.5 * tw_
        ty1 = tyc + 0.5 * th_
        area_s = (sx1 - sx0) * (sy1 - sy0)
        area_t = (tx1 - tx0) * (ty1 - ty0)
        iw = jnp.clip(jnp.minimum(sx1, tx1) - jnp.maximum(sx0, tx0), 0.0)
        ih = jnp.clip(jnp.minimum(sy1, ty1) - jnp.maximum(sy0, ty0), 0.0)
        inter = iw * ih
        union = area_s + area_t - inter
        iou = inter / union
        ew = jnp.clip(jnp.maximum(sx1, tx1) - jnp.minimum(sx0, tx0), 0.0)
        eh = jnp.clip(jnp.maximum(sy1, ty1) - jnp.minimum(sy0, ty0), 0.0)
        earea = ew * eh
        giou = iou - (earea - union) / earea
        s = jnp.clip(giou, 0.0)

        pl_l = _sigmoid(xl)
        sp = jnp.log1p(jnp.exp(-jnp.abs(xl)))
        bce_t = jnp.maximum(xl, 0.0) - xl * s + sp
        bce_0 = jnp.maximum(xl, 0.0) + sp
        corr_sum = corr_sum + jnp.sum(s * bce_t - VFL_A * pl_l * pl_l * bce_0)
        bbox_sum = bbox_sum + jnp.sum(jnp.abs(sbT - tbT))
        giou_sum = giou_sum + jnp.sum(1.0 - giou)

    lane = lax.broadcasted_iota(jnp.int32, (1, 128), 1)
    accvec = (jnp.where(lane == 0, corr_sum, 0.0)
              + jnp.where(lane == 1, bbox_sum, 0.0)
              + jnp.where(lane == 2, giou_sum, 0.0))

    @pl.when(b == 0)
    def _():
        acc_ref[...] = jnp.zeros_like(acc_ref)

    acc_ref[...] += accvec


def _make_sc_solver():
    mesh = plsc.VectorSubcoreMesh(core_axis_name="c", subcore_axis_name="s")
    return pl.kernel(
        _sc_solver_body,
        out_type=jax.ShapeDtypeStruct((B, NT), jnp.int32),
        mesh=mesh,
        compiler_params=pltpu.CompilerParams(needs_layout_passes=False),
        scratch_types=[
            pltpu.VMEM((NT, MPAD), jnp.float32),
            pltpu.VMEM((MPAD,), jnp.float32),
            pltpu.VMEM((MPAD,), jnp.float32),
            pltpu.VMEM((MPAD,), jnp.float32),
            pltpu.VMEM((MPAD,), jnp.int32),
            pltpu.VMEM((MPAD,), jnp.int32),
            pltpu.VMEM((NT,), jnp.int32),
            pltpu.SMEM((NQ + 1,), jnp.int32),
        ],
    )


def kernel(pred_logits, pred_boxes, tgt_labels, tgt_boxes):
    pbT = pred_boxes.transpose(0, 2, 1)
    tbT = tgt_boxes.transpose(0, 2, 1)
    lab3 = tgt_labels.reshape(B, 1, NT)
    nsteps = B // IMS

    cost, dense = pl.pallas_call(
        _cost_dense_body,
        grid=(nsteps,),
        in_specs=[
            pl.BlockSpec((IMS, NQ, NC_CLS), lambda b: (b, 0, 0)),
            pl.BlockSpec((IMS, 4, NQ), lambda b: (b, 0, 0)),
            pl.BlockSpec((IMS, 1, NT), lambda b: (b, 0, 0)),
            pl.BlockSpec((IMS, 4, NT), lambda b: (b, 0, 0)),
        ],
        out_specs=[
            pl.BlockSpec((IMS, NT, MPAD), lambda b: (b, 0, 0)),
            pl.BlockSpec((1, 128), lambda b: (0, 0)),
        ],
        out_shape=[
            jax.ShapeDtypeStruct((B, NT, MPAD), jnp.float32),
            jax.ShapeDtypeStruct((1, 128), jnp.float32),
        ],
    )(pred_logits, pbT, lab3, tbT)

    assign = _make_sc_solver()(cost)

    acc = pl.pallas_call(
        _corr_body,
        grid=(nsteps,),
        in_specs=[
            pl.BlockSpec((IMS, NQ, NC_CLS), lambda b: (b, 0, 0)),
            pl.BlockSpec((IMS, 4, NQ), lambda b: (b, 0, 0)),
            pl.BlockSpec((IMS, 1, NT), lambda b: (b, 0, 0)),
            pl.BlockSpec((IMS, 4, NT), lambda b: (b, 0, 0)),
            pl.BlockSpec((IMS, 1, NT), lambda b: (b, 0, 0)),
        ],
        out_specs=pl.BlockSpec((1, 128), lambda b: (0, 0)),
        out_shape=jax.ShapeDtypeStruct((1, 128), jnp.float32),
    )(pred_logits, pbT, lab3, tbT, assign.reshape(B, 1, NT))

    num_boxes = float(B * NT)
    loss_vfl = (dense[0, 0] + acc[0, 0]) / num_boxes
    loss_bbox = acc[0, 1] / num_boxes
    loss_giou = acc[0, 2] / num_boxes
    total = W_VFL_L * loss_vfl + W_BBOX_L * loss_bbox + W_GIOU_L * loss_giou
    return loss_vfl, loss_bbox, loss_giou, total


# speedup vs baseline: 1.0019x; 1.0019x over previous
import jax
import jax.numpy as jnp
from jax import lax
from jax.experimental import pallas as pl
from jax.experimental.pallas import tpu as pltpu
from jax.experimental.pallas import tpu_sc as plsc

NC_CLS = 80
VFL_A = 0.75
M_ALPHA = 0.25
W_BBOX_C, W_CLASS_C, W_GIOU_C = 5.0, 2.0, 2.0
W_VFL_L, W_BBOX_L, W_GIOU_L = 1.0, 5.0, 2.0
B, NQ, NT = 16, 300, 32
MPAD = 304
NCH = MPAD // 16
INF = 1e18
PADVAL = 1e9


def _sigmoid(x):
    return 1.0 / (1.0 + jnp.exp(-x))


def _xyxy(bT):
    cx, cy, w, h = bT[0:1, :], bT[1:2, :], bT[2:3, :], bT[3:4, :]
    return cx - 0.5 * w, cy - 0.5 * h, cx + 0.5 * w, cy + 0.5 * h


IMS = 4


def _cost_dense_body(logits_ref, pb_ref, lab_ref, tb_ref, cost_ref, dense_ref):
    b = pl.program_id(0)
    dsum = jnp.float32(0.0)
    for im in range(IMS):
        x = logits_ref[im]
        prob = _sigmoid(x)
        neg_cost = (1.0 - M_ALPHA) * (prob * prob) * (-jnp.log(1.0 - prob + 1e-8))
        pos_cost = M_ALPHA * ((1.0 - prob) * (1.0 - prob)) * (-jnp.log(prob + 1e-8))
        diff = pos_cost - neg_cost

        lab = lab_ref[im, 0, :]
        cls_iota = lax.broadcasted_iota(jnp.int32, (NT, NC_CLS), 1)
        onehot = (lab[:, None] == cls_iota).astype(jnp.float32)
        cost_class = lax.dot_general(
            onehot, diff, (((1,), (1,)), ((), ())),
            preferred_element_type=jnp.float32)

        pbT = pb_ref[im]
        tbT = tb_ref[im]
        cost_bbox = jnp.zeros((NT, NQ), jnp.float32)
        for k in range(4):
            cost_bbox = cost_bbox + jnp.abs(tbT[k:k + 1, :].T - pbT[k:k + 1, :])

        tcx, tcy, tw, th = (tbT[0:1, :].T, tbT[1:2, :].T,
                            tbT[2:3, :].T, tbT[3:4, :].T)
        tx0, ty0 = tcx - 0.5 * tw, tcy - 0.5 * th
        tx1, ty1 = tcx + 0.5 * tw, tcy + 0.5 * th
        px0, py0, px1, py1 = _xyxy(pbT)
        area_t = (tx1 - tx0) * (ty1 - ty0)
        area_p = (px1 - px0) * (py1 - py0)
        iw = jnp.clip(jnp.minimum(tx1, px1) - jnp.maximum(tx0, px0), 0.0)
        ih = jnp.clip(jnp.minimum(ty1, py1) - jnp.maximum(ty0, py0), 0.0)
        inter = iw * ih
        union = area_t + area_p - inter
        iou = inter / union
        ew = jnp.clip(jnp.maximum(tx1, px1) - jnp.minimum(tx0, px0), 0.0)
        eh = jnp.clip(jnp.maximum(ty1, py1) - jnp.minimum(ty0, py0), 0.0)
        earea = ew * eh
        giou = iou - (earea - union) / earea

        cmat = W_BBOX_C * cost_bbox + W_CLASS_C * cost_class + W_GIOU_C * (-giou)
        cost_ref[im] = jnp.concatenate(
            [cmat, jnp.full((NT, MPAD - NQ), PADVAL, jnp.float32)], axis=1)

        bce0 = jnp.maximum(x, 0.0) + jnp.log1p(jnp.exp(-jnp.abs(x)))
        dsum = dsum + jnp.sum(VFL_A * prob * prob * bce0)

    lane = lax.broadcasted_iota(jnp.int32, (1, 128), 1)
    dvec = jnp.where(lane == 0, dsum, 0.0)

    @pl.when(b == 0)
    def _():
        dense_ref[...] = jnp.zeros_like(dense_ref)

    dense_ref[...] += dvec


def _sc_solver_body(cost_hbm, out_hbm, cost_v, v_v, w_v, minv_v, way_v,
                    used_v, asn_v, p_s):
    wid = lax.axis_index("s") * 2 + lax.axis_index("c")

    @pl.when(wid < B)
    def _():
        pltpu.sync_copy(cost_hbm.at[wid], cost_v)
        iota16 = lax.broadcasted_iota(jnp.int32, (16,), 0)
        zf = jnp.zeros((16,), jnp.float32)
        zi = jnp.zeros((16,), jnp.int32)
        for k in range(NCH):
            sl = pl.ds(16 * k, 16)
            v_v[sl] = zf
            w_v[sl] = zf
            way_v[sl] = zi

        def clear_p(j, c):
            p_s[j] = jnp.int32(0)
            return c

        lax.fori_loop(0, NQ + 1, clear_p, 0)

        def phase(i, c):
            used_init = ([jnp.zeros((16,), jnp.int32)] * (NCH - 1)
                         + [(iota16 >= (16 - (MPAD - NQ))).astype(jnp.int32)])
            p_s[0] = i
            for k in range(NCH):
                sl = pl.ds(16 * k, 16)
                minv_v[sl] = jnp.full((16,), INF, jnp.float32)
                used_v[sl] = used_init[k]

            def cond(carry):
                return carry[2] != 0

            def body(carry):
                j0, w0, pj0 = carry
                jj0 = jnp.maximum(j0 - 1, 0)
                jj0_splat = jnp.full((16,), jj0, jnp.int32)
                lane0 = iota16 == 0
                j0_pos = jnp.full((16,), j0, jnp.int32) > 0
                plsc.store_scatter(used_v, [jj0_splat],
                                   jnp.ones((16,), jnp.int32),
                                   mask=lane0 & j0_pos)

                w_at_jj0 = plsc.load_gather(w_v, [jj0_splat])
                w0_splat = jnp.full((16,), w0, jnp.float32)
                u_i0 = jnp.where(j0_pos, w_at_jj0, w0_splat)
                row_splat = jnp.full((16,), pj0 - 1, jnp.int32)
                j0_splat = jnp.full((16,), j0, jnp.int32)
                bestv = jnp.full((16,), INF, jnp.float32)
                bestj = jnp.zeros((16,), jnp.int32)
                for k in range(NCH):
                    sl = pl.ds(16 * k, 16)
                    crow = plsc.load_gather(cost_v, [row_splat, iota16 + 16 * k])
                    cu = (crow - u_i0) - v_v[sl]
                    freek = used_v[sl] == 0
                    mv = minv_v[sl]
                    upd = freek & (cu < mv)
                    mv = jnp.where(upd, cu, mv)
                    minv_v[sl] = mv
                    way_v[sl] = jnp.where(upd, j0_splat, way_v[sl])
                    masked = jnp.where(freek, mv, INF)
                    better = masked < bestv
                    bestv = jnp.where(better, masked, bestv)
                    bestj = jnp.where(better, iota16 + (16 * k + 1), bestj)
                delta = jnp.min(bestv)
                deltav = jnp.full((16,), delta, jnp.float32)
                zerov = jnp.zeros((16,), jnp.float32)
                j1 = jnp.min(jnp.where(bestv == deltav, bestj,
                                       jnp.full((16,), 1 << 30, jnp.int32)))
                for k in range(NCH):
                    sl = pl.ds(16 * k, 16)
                    um = used_v[sl] != 0
                    dmask = jnp.where(um, deltav, zerov)
                    w_v[sl] = w_v[sl] + dmask
                    v_v[sl] = v_v[sl] - dmask
                    minv_v[sl] = minv_v[sl] - deltav
                return (j1, w0 + delta, p_s[j1])

            j0f, w0f, _ = lax.while_loop(
                cond, body, (jnp.int32(0), jnp.float32(0.0), i))

            lane0 = iota16 == 0

            def abody(j0):
                jj0_splat = jnp.full((16,), j0 - 1, jnp.int32)
                j1 = jnp.min(plsc.load_gather(way_v, [jj0_splat]))
                row = p_s[j1]
                p_s[j0] = row
                plsc.store_scatter(asn_v, [jnp.full((16,), row - 1, jnp.int32)],
                                   jnp.full((16,), j0 - 1, jnp.int32),
                                   mask=lane0)
                jj1_splat = jnp.full((16,), jnp.maximum(j1 - 1, 0), jnp.int32)
                wsrc = jnp.min(plsc.load_gather(w_v, [jj1_splat]))
                wnew = jnp.where(j1 == 0, w0f, wsrc)
                plsc.store_scatter(w_v, [jj0_splat],
                                   jnp.full((16,), wnew, jnp.float32),
                                   mask=lane0)
                return j1

            lax.while_loop(lambda j0: j0 != 0, abody, j0f)
            return c

        lax.fori_loop(1, NT + 1, phase, 0)
        pltpu.sync_copy(asn_v, out_hbm.at[wid])


def _corr_body(logits_ref, pb_ref, lab_ref, tb_ref, asn_ref, acc_ref):
    b = pl.program_id(0)
    corr_sum = jnp.float32(0.0)
    bbox_sum = jnp.float32(0.0)
    giou_sum = jnp.float32(0.0)
    for im in range(IMS):
        x = logits_ref[im]
        q = asn_ref[im, 0, :]
        lab = lab_ref[im, 0, :]

        q_iota = lax.broadcasted_iota(jnp.int32, (NT, NQ), 1)
        ohq = (q[:, None] == q_iota).astype(jnp.float32)
        xl_rows = lax.dot_general(
            ohq, x, (((1,), (0,)), ((), ())),
            preferred_element_type=jnp.float32)
        cls_iota = lax.broadcasted_iota(jnp.int32, (NT, NC_CLS), 1)
        ohl = (lab[:, None] == cls_iota).astype(jnp.float32)
        xl = jnp.sum(xl_rows * ohl, axis=1)[:, None]

        pbT = pb_ref[im]
        tbT = tb_ref[im]
        sbT = lax.dot_general(
            pbT, ohq, (((1,), (1,)), ((), ())),
            preferred_element_type=jnp.float32)
        sc_, sy_, sw_, sh_ = (sbT[0:1, :].T, sbT[1:2, :].T,
                              sbT[2:3, :].T, sbT[3:4, :].T)
        tc_, tyc, tw_, th_ = (tbT[0:1, :].T, tbT[1:2, :].T,
                              tbT[2:3, :].T, tbT[3:4, :].T)

        sx0 = sc_ - 0.5 * sw_
        sy0 = sy_ - 0.5 * sh_
        sx1 = sc_ + 0.5 * sw_
        sy1 = sy_ + 0.5 * sh_
        tx0 = tc_ - 0.5 * tw_
        ty0 = tyc - 0.5 * th_
        tx1 = tc_ + 0.5 * tw_
        ty1 = tyc + 0.5 * th_
        area_s = (sx1 - sx0) * (sy1 - sy0)
        area_t = (tx1 - tx0) * (ty1 - ty0)
        iw = jnp.clip(jnp.minimum(sx1, tx1) - jnp.maximum(sx0, tx0), 0.0)
        ih = jnp.clip(jnp.minimum(sy1, ty1) - jnp.maximum(sy0, ty0), 0.0)
        inter = iw * ih
        union = area_s + area_t - inter
        iou = inter / union
        ew = jnp.clip(jnp.maximum(sx1, tx1) - jnp.minimum(sx0, tx0), 0.0)
        eh = jnp.clip(jnp.maximum(sy1, ty1) - jnp.minimum(sy0, ty0), 0.0)
        earea = ew * eh
        giou = iou - (earea - union) / earea
        s = jnp.clip(giou, 0.0)

        pl_l = _sigmoid(xl)
        sp = jnp.log1p(jnp.exp(-jnp.abs(xl)))
        bce_t = jnp.maximum(xl, 0.0) - xl * s + sp
        bce_0 = jnp.maximum(xl, 0.0) + sp
        corr_sum = corr_sum + jnp.sum(s * bce_t - VFL_A * pl_l * pl_l * bce_0)
        bbox_sum = bbox_sum + jnp.sum(jnp.abs(sbT - tbT))
        giou_sum = giou_sum + jnp.sum(1.0 - giou)

    lane = lax.broadcasted_iota(jnp.int32, (1, 128), 1)
    accvec = (jnp.where(lane == 0, corr_sum, 0.0)
              + jnp.where(lane == 1, bbox_sum, 0.0)
              + jnp.where(lane == 2, giou_sum, 0.0))

    @pl.when(b == 0)
    def _():
        acc_ref[...] = jnp.zeros_like(acc_ref)

    acc_ref[...] += accvec


def _make_sc_solver():
    mesh = plsc.VectorSubcoreMesh(core_axis_name="c", subcore_axis_name="s")
    return pl.kernel(
        _sc_solver_body,
        out_type=jax.ShapeDtypeStruct((B, NT), jnp.int32),
        mesh=mesh,
        compiler_params=pltpu.CompilerParams(needs_layout_passes=False),
        scratch_types=[
            pltpu.VMEM((NT, MPAD), jnp.float32),
            pltpu.VMEM((MPAD,), jnp.float32),
            pltpu.VMEM((MPAD,), jnp.float32),
            pltpu.VMEM((MPAD,), jnp.float32),
            pltpu.VMEM((MPAD,), jnp.int32),
            pltpu.VMEM((MPAD,), jnp.int32),
            pltpu.VMEM((NT,), jnp.int32),
            pltpu.SMEM((NQ + 1,), jnp.int32),
        ],
    )


def kernel(pred_logits, pred_boxes, tgt_labels, tgt_boxes):
    pbT = pred_boxes.transpose(0, 2, 1)
    tbT = tgt_boxes.transpose(0, 2, 1)
    lab3 = tgt_labels.reshape(B, 1, NT)
    nsteps = B // IMS

    cost, dense = pl.pallas_call(
        _cost_dense_body,
        grid=(nsteps,),
        in_specs=[
            pl.BlockSpec((IMS, NQ, NC_CLS), lambda b: (b, 0, 0)),
            pl.BlockSpec((IMS, 4, NQ), lambda b: (b, 0, 0)),
            pl.BlockSpec((IMS, 1, NT), lambda b: (b, 0, 0)),
            pl.BlockSpec((IMS, 4, NT), lambda b: (b, 0, 0)),
        ],
        out_specs=[
            pl.BlockSpec((IMS, NT, MPAD), lambda b: (b, 0, 0)),
            pl.BlockSpec((1, 128), lambda b: (0, 0)),
        ],
        out_shape=[
            jax.ShapeDtypeStruct((B, NT, MPAD), jnp.float32),
            jax.ShapeDtypeStruct((1, 128), jnp.float32),
        ],
    )(pred_logits, pbT, lab3, tbT)

    assign = _make_sc_solver()(cost)

    acc = pl.pallas_call(
        _corr_body,
        grid=(nsteps,),
        in_specs=[
            pl.BlockSpec((IMS, NQ, NC_CLS), lambda b: (b, 0, 0)),
            pl.BlockSpec((IMS, 4, NQ), lambda b: (b, 0, 0)),
            pl.BlockSpec((IMS, 1, NT), lambda b: (b, 0, 0)),
            pl.BlockSpec((IMS, 4, NT), lambda b: (b, 0, 0)),
            pl.BlockSpec((IMS, 1, NT), lambda b: (b, 0, 0)),
        ],
        out_specs=pl.BlockSpec((1, 128), lambda b: (0, 0)),
        out_shape=jax.ShapeDtypeStruct((1, 128), jnp.float32),
    )(pred_logits, pbT, lab3, tbT, assign.reshape(B, 1, NT))

    num_boxes = float(B * NT)
    loss_vfl = (dense[0, 0] + acc[0, 0]) / num_boxes
    loss_bbox = acc[0, 1] / num_boxes
    loss_giou = acc[0, 2] / num_boxes
    total = W_VFL_L * loss_vfl + W_BBOX_L * loss_bbox + W_GIOU_L * loss_giou
    return loss_vfl, loss_bbox, loss_giou, total
